# baseline (speedup 1.0000x reference)
import functools
import math

import jax
import jax.numpy as jnp
from jax.experimental import pallas as pl
from jax.experimental.pallas import tpu as pltpu

N = 10000
B = 200
HIDDEN = 128
N_RBF = 64
N_BLOCKS = 5
CUTOFF = 5.0
N_TYPES = 100
E_MAX = 2 * (N * N) // B
CHUNK = 1000

EK = 8192
NE = -(-E_MAX // EK)
E_PAD = NE * EK
NT = 1024
N_PAD = -(-N // NT) * NT
NN = N_PAD // NT


def _silu(x):
    return x * jax.nn.sigmoid(x)


def _graph(pos, batch):
    n = pos.shape[0]
    n_chunks = n // CHUNK

    def _chunk_mask(args):
        pb, bb = args
        diff = pb[:, None, :] - pos[None, :, :]
        dist = jnp.sqrt(jnp.sum(diff * diff, axis=-1))
        return (dist > 0.0) & (dist <= CUTOFF) & (bb[:, None] == batch[None, :])

    mask = jax.lax.map(
        _chunk_mask,
        (pos.reshape(n_chunks, CHUNK, 3), batch.reshape(n_chunks, CHUNK)),
    ).reshape(n, n)
    src, dst = jnp.nonzero(mask, size=E_MAX, fill_value=0)
    diff_e = pos[src] - pos[dst]
    d = jnp.sqrt(jnp.sum(diff_e * diff_e, axis=-1))
    valid = mask[src, dst]
    return src, dst, d, valid


def _edge_body(d_ref, vld_ref, xg_ref, gam_ref, cent_ref, wf1_ref, bf1_ref,
               wf2_ref, bf2_ref, wd_ref, out_ref):
    d = d_ref[...]
    gam = gam_ref[0, 0]
    rbf = jnp.exp(-gam * (d - cent_ref[...]) ** 2)
    h = jnp.dot(rbf, wf1_ref[...], preferred_element_type=jnp.float32)
    h = _silu(h + bf1_ref[...])
    w = jnp.dot(h, wf2_ref[...], preferred_element_type=jnp.float32)
    w = w + bf2_ref[...]
    xc = jnp.clip(d / CUTOFF, 0.0, 1.0)
    env = 0.5 * (jnp.cos(jnp.pi * xc) + 1.0) * (xc < 1.0).astype(jnp.float32)
    hd = jnp.dot(xg_ref[...], wd_ref[...], preferred_element_type=jnp.float32)
    out_ref[...] = hd * w * (env * vld_ref[...])


def _edge_call(d2, vld2, xg, gam, cent, wf1t, bf1, wf2t, bf2, wdt):
    zero2 = lambda e: (0, 0)
    return pl.pallas_call(
        _edge_body,
        grid=(NE,),
        in_specs=[
            pl.BlockSpec((EK, 1), lambda e: (e, 0)),
            pl.BlockSpec((EK, 1), lambda e: (e, 0)),
            pl.BlockSpec((EK, HIDDEN), lambda e: (e, 0)),
            pl.BlockSpec((1, 1), zero2),
            pl.BlockSpec((1, N_RBF), zero2),
            pl.BlockSpec((N_RBF, HIDDEN), zero2),
            pl.BlockSpec((1, HIDDEN), zero2),
            pl.BlockSpec((HIDDEN, HIDDEN), zero2),
            pl.BlockSpec((1, HIDDEN), zero2),
            pl.BlockSpec((HIDDEN, HIDDEN), zero2),
        ],
        out_specs=pl.BlockSpec((EK, HIDDEN), lambda e: (e, 0)),
        out_shape=jax.ShapeDtypeStruct((E_PAD, HIDDEN), jnp.float32),
    )(d2, vld2, xg, gam, cent, wf1t, bf1, wf2t, bf2, wdt)


def _update_body(v_ref, x_ref, wb_ref, bb_ref, out_ref):
    s = _silu(v_ref[...])
    out_ref[...] = x_ref[...] + jnp.dot(
        s, wb_ref[...], preferred_element_type=jnp.float32) + bb_ref[...]


def _update_call(v, x, wbt, bb):
    zero2 = lambda n: (0, 0)
    return pl.pallas_call(
        _update_body,
        grid=(NN,),
        in_specs=[
            pl.BlockSpec((NT, HIDDEN), lambda n: (n, 0)),
            pl.BlockSpec((NT, HIDDEN), lambda n: (n, 0)),
            pl.BlockSpec((HIDDEN, HIDDEN), zero2),
            pl.BlockSpec((1, HIDDEN), zero2),
        ],
        out_specs=pl.BlockSpec((NT, HIDDEN), lambda n: (n, 0)),
        out_shape=jax.ShapeDtypeStruct((N_PAD, HIDDEN), jnp.float32),
    )(v, x, wbt, bb)


def _pool_body(x_ref, b_ref, t_ref, w1_ref, b1_ref, w2_ref, b2_ref,
               wr1_ref, br1_ref, wr2_ref, br2_ref, y_ref, pooled, counts):
    step = pl.program_id(0)

    @pl.when(step == 0)
    def _init():
        pooled[...] = jnp.zeros_like(pooled)
        counts[...] = jnp.zeros_like(counts)

    row_ids = jax.lax.broadcasted_iota(jnp.int32, (B, NT), 0)
    onehot = (row_ids == b_ref[...]).astype(jnp.float32)
    pooled[...] += jnp.dot(onehot, x_ref[...],
                           preferred_element_type=jnp.float32)
    counts[...] += jnp.sum(onehot, axis=1, keepdims=True)

    @pl.when(step == NN - 1)
    def _final():
        cnt = jnp.maximum(counts[...], 1.0)
        pm = pooled[...] / cnt
        fp = jnp.dot(t_ref[...], w1_ref[...],
                     preferred_element_type=jnp.float32) + b1_ref[...]
        fp = jnp.dot(_silu(fp), w2_ref[...],
                     preferred_element_type=jnp.float32) + b2_ref[...]
        g = fp[:, :HIDDEN]
        bt = fp[:, HIDDEN:]
        h = g * pm + bt
        h = _silu(jnp.dot(h, wr1_ref[...],
                          preferred_element_type=jnp.float32) + br1_ref[...])
        y_ref[...] = jnp.dot(h, wr2_ref[...],
                             preferred_element_type=jnp.float32) + br2_ref[...]


def _pool_call(x, batch2, T, w1t, b1, w2t, b2, wr1t, br1, wr2t, br2):
    zero2 = lambda n: (0, 0)
    return pl.pallas_call(
        _pool_body,
        grid=(NN,),
        in_specs=[
            pl.BlockSpec((NT, HIDDEN), lambda n: (n, 0)),
            pl.BlockSpec((1, NT), lambda n: (0, n)),
            pl.BlockSpec((B, 1), zero2),
            pl.BlockSpec((1, HIDDEN), zero2),
            pl.BlockSpec((1, HIDDEN), zero2),
            pl.BlockSpec((HIDDEN, 2 * HIDDEN), zero2),
            pl.BlockSpec((1, 2 * HIDDEN), zero2),
            pl.BlockSpec((HIDDEN, HIDDEN), zero2),
            pl.BlockSpec((1, HIDDEN), zero2),
            pl.BlockSpec((HIDDEN, 1), zero2),
            pl.BlockSpec((1, 1), zero2),
        ],
        out_specs=pl.BlockSpec((B, 1), zero2),
        out_shape=jax.ShapeDtypeStruct((B, 1), jnp.float32),
        scratch_shapes=[
            pltpu.VMEM((B, HIDDEN), jnp.float32),
            pltpu.VMEM((B, 1), jnp.float32),
        ],
    )(x, batch2, T, w1t, b1, w2t, b2, wr1t, br1, wr2t, br2)


@jax.jit
def _impl(z, pos, batch, T, emb, gamma, Wf1, bf1, Wf2, bf2, Wd, Wb, bb,
          Wfilm1, bfilm1, Wfilm2, bfilm2, Wr1, br1, Wr2, br2):
    src, dst, d, valid = _graph(pos, batch)

    dp = jnp.zeros((E_PAD,), jnp.float32).at[:E_MAX].set(d)
    vp = jnp.zeros((E_PAD,), jnp.float32).at[:E_MAX].set(
        valid.astype(jnp.float32))
    d2 = dp[:, None]
    vld2 = vp[:, None]
    cent = jnp.linspace(0.0, CUTOFF, N_RBF, dtype=jnp.float32)[None, :]

    x = jnp.zeros((N_PAD, HIDDEN), jnp.float32).at[:N].set(emb[z])

    for i in range(N_BLOCKS):
        xg = x[src]
        xg = jnp.zeros((E_PAD, HIDDEN), jnp.float32).at[:E_MAX].set(xg)
        m = _edge_call(d2, vld2, xg, gamma[i].reshape(1, 1), cent,
                       Wf1[i].T, bf1[i][None, :], Wf2[i].T, bf2[i][None, :],
                       Wd[i].T)
        v = jnp.zeros((N_PAD, HIDDEN), jnp.float32).at[dst].add(m[:E_MAX])
        x = _update_call(v, x, Wb[i].T, bb[i][None, :])

    batch2 = jnp.full((1, N_PAD), B, jnp.int32).at[0, :N].set(
        batch.astype(jnp.int32))
    y = _pool_call(x, batch2, T, Wfilm1.T, bfilm1[None, :], Wfilm2.T,
                   bfilm2[None, :], Wr1.T, br1[None, :], Wr2.T,
                   br2[None, :])
    return y[:, 0]


def kernel(z, pos, batch, T, emb, gamma, Wf1, bf1, Wf2, bf2, Wd, Wb, bb,
           Wfilm1, bfilm1, Wfilm2, bfilm2, Wr1, br1, Wr2, br2):
    return _impl(z, pos, batch, T, emb, gamma, Wf1, bf1, Wf2, bf2, Wd, Wb, bb,
                 Wfilm1, bfilm1, Wfilm2, bfilm2, Wr1, br1, Wr2, br2)


# speedup vs baseline: 2.6787x; 2.6787x over previous
import jax
import jax.numpy as jnp
from jax.experimental import pallas as pl
from jax.experimental.pallas import tpu as pltpu

N = 10000
B = 200
HIDDEN = 128
N_RBF = 64
N_BLOCKS = 5
CUTOFF = 5.0

TI = 8
TJ = 128
NT_POOL = 1024
N_PAD = -(-N // NT_POOL) * NT_POOL
NN = N_PAD // NT_POOL
N_TILES = N_PAD // TI


def _silu(x):
    return x * jax.nn.sigmoid(x)


def _conv_body(bounds_ref, gam_ref, pos_ref, bat_ref, x_ref, cent_ref,
               wf1_ref, bf1_ref, wf2_ref, bf2_ref, wd_ref, out_ref, acc_ref):
    t = pl.program_id(0)
    a0 = t * TI
    jlo = bounds_ref[0, t]
    jhi = bounds_ref[1, t]
    gam = gam_ref[0, 0]
    acc_ref[...] = jnp.zeros_like(acc_ref)
    cent = cent_ref[...]

    def jloop(jt, carry):
        j0 = jt * TJ
        pj = pos_ref[pl.dslice(j0, TJ), :]
        bj = bat_ref[pl.dslice(j0, TJ), :]
        xj = x_ref[pl.dslice(j0, TJ), :]
        hd = jnp.dot(xj, wd_ref[...],
                     preferred_element_type=jnp.float32)
        for ii in range(TI):
            pi = pos_ref[pl.dslice(a0 + ii, 1), :]
            bi = bat_ref[pl.dslice(a0 + ii, 1), :]
            diff = pj - pi
            d = jnp.sqrt(jnp.sum(diff * diff, axis=1, keepdims=True))
            rbf = jnp.exp(-gam * (d - cent) ** 2)
            h = _silu(jnp.dot(rbf, wf1_ref[...],
                              preferred_element_type=jnp.float32)
                      + bf1_ref[...])
            w = jnp.dot(h, wf2_ref[...],
                        preferred_element_type=jnp.float32) + bf2_ref[...]
            xc = jnp.clip(d / CUTOFF, 0.0, 1.0)
            env = 0.5 * (jnp.cos(jnp.pi * xc) + 1.0) * (
                xc < 1.0).astype(jnp.float32)
            msk = ((d > 0.0) & (d <= CUTOFF)
                   & (bj == bi)).astype(jnp.float32)
            contrib = jnp.sum(hd * w * (env * msk), axis=0, keepdims=True)
            acc_ref[pl.dslice(ii, 1), :] += contrib
        return carry

    jax.lax.fori_loop(jlo, jhi, jloop, 0)
    out_ref[...] = acc_ref[...]


def _conv_call(bounds, gam, posp, batp, x, cent, wf1t, bf1, wf2t, bf2, wdt):
    whole = lambda t: (0, 0)
    return pl.pallas_call(
        _conv_body,
        grid=(N_TILES,),
        in_specs=[
            pl.BlockSpec(memory_space=pltpu.SMEM),
            pl.BlockSpec(memory_space=pltpu.SMEM),
            pl.BlockSpec((N_PAD, 8), whole),
            pl.BlockSpec((N_PAD, 1), whole),
            pl.BlockSpec((N_PAD, HIDDEN), whole),
            pl.BlockSpec((1, N_RBF), whole),
            pl.BlockSpec((N_RBF, HIDDEN), whole),
            pl.BlockSpec((1, HIDDEN), whole),
            pl.BlockSpec((HIDDEN, HIDDEN), whole),
            pl.BlockSpec((1, HIDDEN), whole),
            pl.BlockSpec((HIDDEN, HIDDEN), whole),
        ],
        out_specs=pl.BlockSpec((TI, HIDDEN), lambda t: (t, 0)),
        out_shape=jax.ShapeDtypeStruct((N_PAD, HIDDEN), jnp.float32),
        scratch_shapes=[pltpu.VMEM((TI, HIDDEN), jnp.float32)],
    )(bounds, gam, posp, batp, x, cent, wf1t, bf1, wf2t, bf2, wdt)


def _update_body(v_ref, x_ref, wb_ref, bb_ref, out_ref):
    s = _silu(v_ref[...])
    out_ref[...] = x_ref[...] + jnp.dot(
        s, wb_ref[...], preferred_element_type=jnp.float32) + bb_ref[...]


def _update_call(v, x, wbt, bb):
    zero2 = lambda n: (0, 0)
    return pl.pallas_call(
        _update_body,
        grid=(NN,),
        in_specs=[
            pl.BlockSpec((NT_POOL, HIDDEN), lambda n: (n, 0)),
            pl.BlockSpec((NT_POOL, HIDDEN), lambda n: (n, 0)),
            pl.BlockSpec((HIDDEN, HIDDEN), zero2),
            pl.BlockSpec((1, HIDDEN), zero2),
        ],
        out_specs=pl.BlockSpec((NT_POOL, HIDDEN), lambda n: (n, 0)),
        out_shape=jax.ShapeDtypeStruct((N_PAD, HIDDEN), jnp.float32),
    )(v, x, wbt, bb)


def _pool_body(x_ref, b_ref, t_ref, w1_ref, b1_ref, w2_ref, b2_ref,
               wr1_ref, br1_ref, wr2_ref, br2_ref, y_ref, pooled, counts):
    step = pl.program_id(0)

    @pl.when(step == 0)
    def _init():
        pooled[...] = jnp.zeros_like(pooled)
        counts[...] = jnp.zeros_like(counts)

    row_ids = jax.lax.broadcasted_iota(jnp.int32, (B, NT_POOL), 0)
    onehot = (row_ids == b_ref[...]).astype(jnp.float32)
    pooled[...] += jnp.dot(onehot, x_ref[...],
                           preferred_element_type=jnp.float32)
    counts[...] += jnp.sum(onehot, axis=1, keepdims=True)

    @pl.when(step == NN - 1)
    def _final():
        cnt = jnp.maximum(counts[...], 1.0)
        pm = pooled[...] / cnt
        fp = jnp.dot(t_ref[...], w1_ref[...],
                     preferred_element_type=jnp.float32) + b1_ref[...]
        fp = jnp.dot(_silu(fp), w2_ref[...],
                     preferred_element_type=jnp.float32) + b2_ref[...]
        g = fp[:, :HIDDEN]
        bt = fp[:, HIDDEN:]
        h = g * pm + bt
        h = _silu(jnp.dot(h, wr1_ref[...],
                          preferred_element_type=jnp.float32) + br1_ref[...])
        y_ref[...] = jnp.dot(h, wr2_ref[...],
                             preferred_element_type=jnp.float32) + br2_ref[...]


def _pool_call(x, batch2, T, w1t, b1, w2t, b2, wr1t, br1, wr2t, br2):
    zero2 = lambda n: (0, 0)
    return pl.pallas_call(
        _pool_body,
        grid=(NN,),
        in_specs=[
            pl.BlockSpec((NT_POOL, HIDDEN), lambda n: (n, 0)),
            pl.BlockSpec((1, NT_POOL), lambda n: (0, n)),
            pl.BlockSpec((B, 1), zero2),
            pl.BlockSpec((1, HIDDEN), zero2),
            pl.BlockSpec((1, HIDDEN), zero2),
            pl.BlockSpec((HIDDEN, 2 * HIDDEN), zero2),
            pl.BlockSpec((1, 2 * HIDDEN), zero2),
            pl.BlockSpec((HIDDEN, HIDDEN), zero2),
            pl.BlockSpec((1, HIDDEN), zero2),
            pl.BlockSpec((HIDDEN, 1), zero2),
            pl.BlockSpec((1, 1), zero2),
        ],
        out_specs=pl.BlockSpec((B, 1), zero2),
        out_shape=jax.ShapeDtypeStruct((B, 1), jnp.float32),
        scratch_shapes=[
            pltpu.VMEM((B, HIDDEN), jnp.float32),
            pltpu.VMEM((B, 1), jnp.float32),
        ],
    )(x, batch2, T, w1t, b1, w2t, b2, wr1t, br1, wr2t, br2)


@jax.jit
def _impl(z, pos, batch, T, emb, gamma, Wf1, bf1, Wf2, bf2, Wd, Wb, bb,
          Wfilm1, bfilm1, Wfilm2, bfilm2, Wr1, br1, Wr2, br2):
    batch = batch.astype(jnp.int32)
    batp = jnp.full((N_PAD, 1), B, jnp.int32).at[:N, 0].set(batch)
    posp = jnp.zeros((N_PAD, 8), jnp.float32).at[:N, :3].set(pos)
    x = jnp.zeros((N_PAD, HIDDEN), jnp.float32).at[:N].set(emb[z])
    cent = jnp.linspace(0.0, CUTOFF, N_RBF, dtype=jnp.float32)[None, :]

    bflat = batp[:, 0]
    first = bflat[0::TI]
    last = bflat[TI - 1::TI]
    lo_atom = jnp.searchsorted(bflat, first, side='left')
    hi_atom = jnp.searchsorted(bflat, last, side='right')
    jlo = (lo_atom // TJ).astype(jnp.int32)
    jhi = (-(-hi_atom // TJ)).astype(jnp.int32)
    bounds = jnp.stack([jlo, jhi], axis=0)

    for i in range(N_BLOCKS):
        v = _conv_call(bounds, gamma[i].reshape(1, 1), posp, batp, x, cent,
                       Wf1[i].T, bf1[i][None, :], Wf2[i].T, bf2[i][None, :],
                       Wd[i].T)
        x = _update_call(v, x, Wb[i].T, bb[i][None, :])

    batch2 = jnp.full((1, N_PAD), B, jnp.int32).at[0, :N].set(batch)
    y = _pool_call(x, batch2, T, Wfilm1.T, bfilm1[None, :], Wfilm2.T,
                   bfilm2[None, :], Wr1.T, br1[None, :], Wr2.T,
                   br2[None, :])
    return y[:, 0]


def kernel(z, pos, batch, T, emb, gamma, Wf1, bf1, Wf2, bf2, Wd, Wb, bb,
           Wfilm1, bfilm1, Wfilm2, bfilm2, Wr1, br1, Wr2, br2):
    return _impl(z, pos, batch, T, emb, gamma, Wf1, bf1, Wf2, bf2, Wd, Wb, bb,
                 Wfilm1, bfilm1, Wfilm2, bfilm2, Wr1, br1, Wr2, br2)


# speedup vs baseline: 2.6868x; 1.0030x over previous
import jax
import jax.numpy as jnp
from jax.experimental import pallas as pl
from jax.experimental.pallas import tpu as pltpu

N = 10000
B = 200
HIDDEN = 128
N_RBF = 64
N_BLOCKS = 5
CUTOFF = 5.0

TI = 8
TJ = 128
NT_POOL = 1024
N_PAD = -(-N // NT_POOL) * NT_POOL
NN = N_PAD // NT_POOL
N_TILES = N_PAD // TI


def _silu(x):
    return x * jax.nn.sigmoid(x)


def _conv_body(bounds_ref, gam_ref, pos_ref, bat_ref, x_ref, cent_ref,
               wf1_ref, bf1_ref, wf2_ref, bf2_ref, wd_ref, out_ref, acc_ref,
               d_big, em_big):
    t = pl.program_id(0)
    a0 = t * TI
    jlo = bounds_ref[0, t]
    jhi = bounds_ref[1, t]
    gam = gam_ref[0, 0]
    acc_ref[...] = jnp.zeros_like(acc_ref)
    cent = cent_ref[...]

    def jloop(jt, carry):
        j0 = jt * TJ
        pj = pos_ref[pl.dslice(j0, TJ), :]
        bj = bat_ref[pl.dslice(j0, TJ), :]
        xj = x_ref[pl.dslice(j0, TJ), :]
        hd = jnp.dot(xj, wd_ref[...],
                     preferred_element_type=jnp.float32)
        for ii in range(TI):
            pi = pos_ref[pl.dslice(a0 + ii, 1), :]
            bi = bat_ref[pl.dslice(a0 + ii, 1), :]
            diff = pj - pi
            d = jnp.sqrt(jnp.sum(diff * diff, axis=1, keepdims=True))
            xc = jnp.clip(d / CUTOFF, 0.0, 1.0)
            env = 0.5 * (jnp.cos(jnp.pi * xc) + 1.0) * (
                xc < 1.0).astype(jnp.float32)
            msk = ((d > 0.0) & (d <= CUTOFF)
                   & (bj == bi)).astype(jnp.float32)
            d_big[pl.dslice(ii * TJ, TJ), :] = d
            em_big[pl.dslice(ii * TJ, TJ), :] = env * msk
        rbf = jnp.exp(-gam * (d_big[...] - cent) ** 2)
        h = _silu(jnp.dot(rbf, wf1_ref[...],
                          preferred_element_type=jnp.float32) + bf1_ref[...])
        w = jnp.dot(h, wf2_ref[...],
                    preferred_element_type=jnp.float32) + bf2_ref[...]
        wm = (w * em_big[...]).reshape(TI, TJ, HIDDEN)
        acc_ref[...] += jnp.sum(wm * hd.reshape(1, TJ, HIDDEN), axis=1)
        return carry

    jax.lax.fori_loop(jlo, jhi, jloop, 0)
    out_ref[...] = acc_ref[...]


def _conv_call(bounds, gam, posp, batp, x, cent, wf1t, bf1, wf2t, bf2, wdt):
    whole = lambda t: (0, 0)
    return pl.pallas_call(
        _conv_body,
        grid=(N_TILES,),
        in_specs=[
            pl.BlockSpec(memory_space=pltpu.SMEM),
            pl.BlockSpec(memory_space=pltpu.SMEM),
            pl.BlockSpec((N_PAD, 8), whole),
            pl.BlockSpec((N_PAD, 1), whole),
            pl.BlockSpec((N_PAD, HIDDEN), whole),
            pl.BlockSpec((1, N_RBF), whole),
            pl.BlockSpec((N_RBF, HIDDEN), whole),
            pl.BlockSpec((1, HIDDEN), whole),
            pl.BlockSpec((HIDDEN, HIDDEN), whole),
            pl.BlockSpec((1, HIDDEN), whole),
            pl.BlockSpec((HIDDEN, HIDDEN), whole),
        ],
        out_specs=pl.BlockSpec((TI, HIDDEN), lambda t: (t, 0)),
        out_shape=jax.ShapeDtypeStruct((N_PAD, HIDDEN), jnp.float32),
        scratch_shapes=[
            pltpu.VMEM((TI, HIDDEN), jnp.float32),
            pltpu.VMEM((TI * TJ, 1), jnp.float32),
            pltpu.VMEM((TI * TJ, 1), jnp.float32),
        ],
    )(bounds, gam, posp, batp, x, cent, wf1t, bf1, wf2t, bf2, wdt)


def _update_body(v_ref, x_ref, wb_ref, bb_ref, out_ref):
    s = _silu(v_ref[...])
    out_ref[...] = x_ref[...] + jnp.dot(
        s, wb_ref[...], preferred_element_type=jnp.float32) + bb_ref[...]


def _update_call(v, x, wbt, bb):
    zero2 = lambda n: (0, 0)
    return pl.pallas_call(
        _update_body,
        grid=(NN,),
        in_specs=[
            pl.BlockSpec((NT_POOL, HIDDEN), lambda n: (n, 0)),
            pl.BlockSpec((NT_POOL, HIDDEN), lambda n: (n, 0)),
            pl.BlockSpec((HIDDEN, HIDDEN), zero2),
            pl.BlockSpec((1, HIDDEN), zero2),
        ],
        out_specs=pl.BlockSpec((NT_POOL, HIDDEN), lambda n: (n, 0)),
        out_shape=jax.ShapeDtypeStruct((N_PAD, HIDDEN), jnp.float32),
    )(v, x, wbt, bb)


def _pool_body(x_ref, b_ref, t_ref, w1_ref, b1_ref, w2_ref, b2_ref,
               wr1_ref, br1_ref, wr2_ref, br2_ref, y_ref, pooled, counts):
    step = pl.program_id(0)

    @pl.when(step == 0)
    def _init():
        pooled[...] = jnp.zeros_like(pooled)
        counts[...] = jnp.zeros_like(counts)

    row_ids = jax.lax.broadcasted_iota(jnp.int32, (B, NT_POOL), 0)
    onehot = (row_ids == b_ref[...]).astype(jnp.float32)
    pooled[...] += jnp.dot(onehot, x_ref[...],
                           preferred_element_type=jnp.float32)
    counts[...] += jnp.sum(onehot, axis=1, keepdims=True)

    @pl.when(step == NN - 1)
    def _final():
        cnt = jnp.maximum(counts[...], 1.0)
        pm = pooled[...] / cnt
        fp = jnp.dot(t_ref[...], w1_ref[...],
                     preferred_element_type=jnp.float32) + b1_ref[...]
        fp = jnp.dot(_silu(fp), w2_ref[...],
                     preferred_element_type=jnp.float32) + b2_ref[...]
        g = fp[:, :HIDDEN]
        bt = fp[:, HIDDEN:]
        h = g * pm + bt
        h = _silu(jnp.dot(h, wr1_ref[...],
                          preferred_element_type=jnp.float32) + br1_ref[...])
        y_ref[...] = jnp.dot(h, wr2_ref[...],
                             preferred_element_type=jnp.float32) + br2_ref[...]


def _pool_call(x, batch2, T, w1t, b1, w2t, b2, wr1t, br1, wr2t, br2):
    zero2 = lambda n: (0, 0)
    return pl.pallas_call(
        _pool_body,
        grid=(NN,),
        in_specs=[
            pl.BlockSpec((NT_POOL, HIDDEN), lambda n: (n, 0)),
            pl.BlockSpec((1, NT_POOL), lambda n: (0, n)),
            pl.BlockSpec((B, 1), zero2),
            pl.BlockSpec((1, HIDDEN), zero2),
            pl.BlockSpec((1, HIDDEN), zero2),
            pl.BlockSpec((HIDDEN, 2 * HIDDEN), zero2),
            pl.BlockSpec((1, 2 * HIDDEN), zero2),
            pl.BlockSpec((HIDDEN, HIDDEN), zero2),
            pl.BlockSpec((1, HIDDEN), zero2),
            pl.BlockSpec((HIDDEN, 1), zero2),
            pl.BlockSpec((1, 1), zero2),
        ],
        out_specs=pl.BlockSpec((B, 1), zero2),
        out_shape=jax.ShapeDtypeStruct((B, 1), jnp.float32),
        scratch_shapes=[
            pltpu.VMEM((B, HIDDEN), jnp.float32),
            pltpu.VMEM((B, 1), jnp.float32),
        ],
    )(x, batch2, T, w1t, b1, w2t, b2, wr1t, br1, wr2t, br2)


@jax.jit
def _impl(z, pos, batch, T, emb, gamma, Wf1, bf1, Wf2, bf2, Wd, Wb, bb,
          Wfilm1, bfilm1, Wfilm2, bfilm2, Wr1, br1, Wr2, br2):
    batch = batch.astype(jnp.int32)
    batp = jnp.full((N_PAD, 1), B, jnp.int32).at[:N, 0].set(batch)
    posp = jnp.zeros((N_PAD, 8), jnp.float32).at[:N, :3].set(pos)
    x = jnp.zeros((N_PAD, HIDDEN), jnp.float32).at[:N].set(emb[z])
    cent = jnp.linspace(0.0, CUTOFF, N_RBF, dtype=jnp.float32)[None, :]

    bflat = batp[:, 0]
    first = bflat[0::TI]
    last = bflat[TI - 1::TI]
    lo_atom = jnp.searchsorted(bflat, first, side='left')
    hi_atom = jnp.searchsorted(bflat, last, side='right')
    jlo = (lo_atom // TJ).astype(jnp.int32)
    jhi = (-(-hi_atom // TJ)).astype(jnp.int32)
    bounds = jnp.stack([jlo, jhi], axis=0)

    for i in range(N_BLOCKS):
        v = _conv_call(bounds, gamma[i].reshape(1, 1), posp, batp, x, cent,
                       Wf1[i].T, bf1[i][None, :], Wf2[i].T, bf2[i][None, :],
                       Wd[i].T)
        x = _update_call(v, x, Wb[i].T, bb[i][None, :])

    batch2 = jnp.full((1, N_PAD), B, jnp.int32).at[0, :N].set(batch)
    y = _pool_call(x, batch2, T, Wfilm1.T, bfilm1[None, :], Wfilm2.T,
                   bfilm2[None, :], Wr1.T, br1[None, :], Wr2.T,
                   br2[None, :])
    return y[:, 0]


def kernel(z, pos, batch, T, emb, gamma, Wf1, bf1, Wf2, bf2, Wd, Wb, bb,
           Wfilm1, bfilm1, Wfilm2, bfilm2, Wr1, br1, Wr2, br2):
    return _impl(z, pos, batch, T, emb, gamma, Wf1, bf1, Wf2, bf2, Wd, Wb, bb,
                 Wfilm1, bfilm1, Wfilm2, bfilm2, Wr1, br1, Wr2, br2)
